# baseline (speedup 1.0000x reference)
import functools

import jax
import jax.numpy as jnp
from jax import lax
from jax.experimental import pallas as pl
from jax.experimental.pallas import tpu as pltpu
from jax.experimental.pallas import tpu_sc as plsc

K = 16


def _topk_body(T, N, nf_ref, emb_ref, idx_ref, flat_ref):
    pid = pl.program_id(0)
    nv1 = nf_ref[0, 0, 0]
    nv2 = nf_ref[1, 0, 0]
    nv3 = jnp.concatenate([nv1, nv2], axis=-1)
    emb = emb_ref[...]
    logits = lax.dot_general(emb, nv3, (((1,), (1,)), ((), ())))

    M = logits.shape[0]
    C = 128
    R = 4
    NCH = N // C
    BIGN = jnp.int32(1 << 20)

    lane = lax.broadcasted_iota(jnp.int32, (M, C), 1)
    sentv = jnp.full((M, C), -1.0, jnp.float32)
    sentn = jnp.full((M, C), BIGN, jnp.int32)
    mxr = logits[:, 0:C]
    for c in range(1, NCH):
        mxr = jnp.maximum(mxr, logits[:, c * C:(c + 1) * C])
    for s in (64, 32, 16, 8, 4, 2, 1):
        mxr = jnp.maximum(mxr, jnp.roll(mxr, s, axis=1))
    vs = [sentv] * R
    ns = [sentn] * R
    for c in range(NCH):
        nc = lane + c * C
        vc = jnp.exp(logits[:, c * C:(c + 1) * C] - mxr)
        bs = [vc > v for v in vs]
        for r in range(R - 1, 0, -1):
            vs[r] = jnp.where(bs[r - 1], vs[r - 1],
                              jnp.where(bs[r], vc, vs[r]))
            ns[r] = jnp.where(bs[r - 1], ns[r - 1],
                              jnp.where(bs[r], nc, ns[r]))
        vs[0] = jnp.where(bs[0], vc, vs[0])
        ns[0] = jnp.where(bs[0], nc, ns[0])

    V = jnp.concatenate([jnp.transpose(v) for v in vs], axis=1)
    Nn = jnp.concatenate([jnp.transpose(n) for n in ns], axis=1)
    S = 8
    NT = C // S
    Vt = [V[i * S:(i + 1) * S] for i in range(NT)]
    Nt = [Nn[i * S:(i + 1) * S] for i in range(NT)]

    nrows = []
    for k in range(K):
        mv, mi = Vt[0], Nt[0]
        for i in range(1, NT):
            take = (Vt[i] > mv) | ((Vt[i] == mv) & (Nt[i] < mi))
            mv = jnp.where(take, Vt[i], mv)
            mi = jnp.where(take, Nt[i], mi)
        for ax, s in ((0, 4), (0, 2), (0, 1), (1, M), (1, 2 * M)):
            rv = jnp.roll(mv, s, axis=ax)
            ri = jnp.roll(mi, s, axis=ax)
            take = (rv > mv) | ((rv == mv) & (ri < mi))
            mv = jnp.where(take, rv, mv)
            mi = jnp.where(take, ri, mi)
        nstar = mi
        nrows.append(nstar)
        for i in range(NT):
            Vt[i] = jnp.where(Nt[i] == nstar, -1.0, Vt[i])

    nmat = jnp.concatenate([nr[0:1] for nr in nrows], axis=0)
    idx_acc = jnp.transpose(nmat[:, :M])
    idx_ref[0] = idx_acc
    flat_ref[0] = idx_acc + pid * N

    exh = jnp.zeros((S, 4 * M), jnp.int32)
    for i in range(NT):
        used = jnp.where(Vt[i] < 0.0, 1, 0)
        u2 = used + jnp.roll(used, M, axis=1)
        u4 = u2 + jnp.roll(u2, 2 * M, axis=1)
        exh = jnp.maximum(exh, u4)
    exhausted = jnp.max(jnp.where(exh >= R, 1, 0))

    G = 8
    @pl.when(exhausted > 0)
    def _slow_path():
        for g in range(M // G):
            lg = logits[g * G:(g + 1) * G, :]
            l = jnp.exp(lg - jnp.max(lg, axis=1, keepdims=True))
            iota_n = lax.broadcasted_iota(jnp.int32, (G, N), 1)
            col = lax.broadcasted_iota(jnp.int32, (G, K), 1)
            idx_acc = jnp.zeros((G, K), jnp.int32)
            for k in range(K):
                mx = jnp.max(l, axis=1, keepdims=True)
                am = jnp.min(jnp.where(l >= mx, iota_n, N), axis=1,
                             keepdims=True)
                idx_acc = jnp.where(col == k, am, idx_acc)
                l = jnp.where(iota_n == am, -1.0, l)
            idx_ref[0, g * G:(g + 1) * G, :] = idx_acc
            flat_ref[0, g * G:(g + 1) * G, :] = idx_acc + pid * N


def _topk_call(nf, emb):
    two, B, T, N, D = nf.shape
    M = emb.shape[0]
    BT = B * T
    return pl.pallas_call(
        functools.partial(_topk_body, T, N),
        grid=(BT,),
        in_specs=[
            pl.BlockSpec((2, 1, 1, N, D), lambda i: (0, i // T, i % T, 0, 0)),
            pl.BlockSpec((M, 2 * D), lambda i: (0, 0)),
        ],
        out_specs=[
            pl.BlockSpec((1, M, K), lambda i: (i, 0, 0)),
            pl.BlockSpec((1, M, K), lambda i: (i, 0, 0)),
        ],
        out_shape=[
            jax.ShapeDtypeStruct((BT, M, K), jnp.int32),
            jax.ShapeDtypeStruct((BT, M, K), jnp.int32),
        ],
    )(nf, emb)


_NW = 32
_CH = 128


def _make_sc_gather(total_rows, D):
    per_w = total_rows // _NW
    nch = per_w // _CH
    mesh = plsc.VectorSubcoreMesh(core_axis_name="c", subcore_axis_name="s")

    @functools.partial(
        pl.kernel,
        out_type=jax.ShapeDtypeStruct((total_rows, D), jnp.float32),
        mesh=mesh,
        scratch_types=[
            pltpu.VMEM((nch, _CH), jnp.int32),
            pltpu.VMEM((_CH, D), jnp.float32),
            pltpu.VMEM((_CH, D), jnp.float32),
            pltpu.SemaphoreType.DMA,
            pltpu.SemaphoreType.DMA,
        ],
    )
    def gather(idx_hbm, table_hbm, out_hbm, idx_v, buf0, buf1, sem0, sem1):
        wid = lax.axis_index("s") * 2 + lax.axis_index("c")
        pltpu.sync_copy(idx_hbm.at[wid], idx_v)
        base = wid * per_w

        pltpu.async_copy(table_hbm.at[idx_v.at[0]], buf0, sem0)

        def step(j, carry):
            c0 = 2 * j
            pltpu.async_copy(table_hbm.at[idx_v.at[c0 + 1]], buf1, sem1)
            pltpu.make_async_copy(table_hbm.at[idx_v.at[c0]], buf0, sem0).wait()
            pltpu.sync_copy(buf0, out_hbm.at[pl.ds(base + c0 * _CH, _CH)])

            @pl.when(j < nch // 2 - 1)
            def _():
                pltpu.async_copy(table_hbm.at[idx_v.at[c0 + 2]], buf0, sem0)

            pltpu.make_async_copy(
                table_hbm.at[idx_v.at[c0 + 1]], buf1, sem1).wait()
            pltpu.sync_copy(buf1,
                            out_hbm.at[pl.ds(base + (c0 + 1) * _CH, _CH)])
            return carry

        lax.fori_loop(0, nch // 2, step, 0)

    return gather


def kernel(node_feature, node_embeddings):
    two, B, T, N, D = node_feature.shape
    M = node_embeddings.shape[0]

    idx, flat1 = _topk_call(node_feature, node_embeddings)
    flat2 = flat1 + B * T * N
    flat = jnp.concatenate([flat1.reshape(-1), flat2.reshape(-1)])
    total_rows = flat.shape[0]

    table = node_feature.reshape(two * B * T * N, D)
    rows = _make_sc_gather(total_rows, D)(
        flat.reshape(_NW, total_rows // (_NW * _CH), _CH), table)
    sel = rows.reshape(2, B, T, M, K, D)

    indices = idx.reshape(B, T, M, K)
    batch_indices = jnp.broadcast_to(
        jnp.arange(B, dtype=indices.dtype).reshape(B, 1, 1, 1), (B, T, M, K))
    time_indices = jnp.broadcast_to(
        jnp.arange(T, dtype=indices.dtype).reshape(1, T, 1, 1), (B, T, M, K))
    return (sel[0], sel[1], batch_indices, time_indices, indices)


# speedup vs baseline: 1.0029x; 1.0029x over previous
import functools

import jax
import jax.numpy as jnp
from jax import lax
from jax.experimental import pallas as pl
from jax.experimental.pallas import tpu as pltpu
from jax.experimental.pallas import tpu_sc as plsc

K = 16


def _topk_body(T, N, nf_ref, emb_ref, idx_ref, flat_ref):
    pid = pl.program_id(0)
    for d in range(2):
        _topk_slice(N, d, 2 * pid + d, nf_ref, emb_ref, idx_ref, flat_ref)


def _topk_slice(N, d, bt, nf_ref, emb_ref, idx_ref, flat_ref):
    nv1 = nf_ref[0, 0, d]
    nv2 = nf_ref[1, 0, d]
    nv3 = jnp.concatenate([nv1, nv2], axis=-1)
    emb = emb_ref[...]
    logits = lax.dot_general(emb, nv3, (((1,), (1,)), ((), ())))

    M = logits.shape[0]
    C = 128
    R = 4
    NCH = N // C
    BIGN = jnp.int32(1 << 20)

    lane = lax.broadcasted_iota(jnp.int32, (M, C), 1)
    sentv = jnp.full((M, C), -1.0, jnp.float32)
    sentn = jnp.full((M, C), BIGN, jnp.int32)
    mxr = logits[:, 0:C]
    for c in range(1, NCH):
        mxr = jnp.maximum(mxr, logits[:, c * C:(c + 1) * C])
    for s in (64, 32, 16, 8, 4, 2, 1):
        mxr = jnp.maximum(mxr, jnp.roll(mxr, s, axis=1))
    vs = [sentv] * R
    ns = [sentn] * R
    for c in range(NCH):
        nc = lane + c * C
        vc = jnp.exp(logits[:, c * C:(c + 1) * C] - mxr)
        bs = [vc > v for v in vs]
        for r in range(R - 1, 0, -1):
            vs[r] = jnp.where(bs[r - 1], vs[r - 1],
                              jnp.where(bs[r], vc, vs[r]))
            ns[r] = jnp.where(bs[r - 1], ns[r - 1],
                              jnp.where(bs[r], nc, ns[r]))
        vs[0] = jnp.where(bs[0], vc, vs[0])
        ns[0] = jnp.where(bs[0], nc, ns[0])

    V = jnp.concatenate([jnp.transpose(v) for v in vs], axis=1)
    Nn = jnp.concatenate([jnp.transpose(n) for n in ns], axis=1)
    S = 8
    NT = C // S
    Vt = [V[i * S:(i + 1) * S] for i in range(NT)]
    Nt = [Nn[i * S:(i + 1) * S] for i in range(NT)]

    nrows = []
    for k in range(K):
        mv, mi = Vt[0], Nt[0]
        for i in range(1, NT):
            take = (Vt[i] > mv) | ((Vt[i] == mv) & (Nt[i] < mi))
            mv = jnp.where(take, Vt[i], mv)
            mi = jnp.where(take, Nt[i], mi)
        for ax, s in ((0, 4), (0, 2), (0, 1), (1, M), (1, 2 * M)):
            rv = jnp.roll(mv, s, axis=ax)
            ri = jnp.roll(mi, s, axis=ax)
            take = (rv > mv) | ((rv == mv) & (ri < mi))
            mv = jnp.where(take, rv, mv)
            mi = jnp.where(take, ri, mi)
        nstar = mi
        nrows.append(nstar)
        for i in range(NT):
            Vt[i] = jnp.where(Nt[i] == nstar, -1.0, Vt[i])

    nmat = jnp.concatenate([nr[0:1] for nr in nrows], axis=0)
    idx_acc = jnp.transpose(nmat[:, :M])
    idx_ref[d] = idx_acc
    flat_ref[d] = idx_acc + bt * N

    exh = jnp.zeros((S, 4 * M), jnp.int32)
    for i in range(NT):
        used = jnp.where(Vt[i] < 0.0, 1, 0)
        u2 = used + jnp.roll(used, M, axis=1)
        u4 = u2 + jnp.roll(u2, 2 * M, axis=1)
        exh = jnp.maximum(exh, u4)
    exhausted = jnp.max(jnp.where(exh >= R, 1, 0))

    G = 8
    @pl.when(exhausted > 0)
    def _slow_path():
        for g in range(M // G):
            lg = logits[g * G:(g + 1) * G, :]
            l = jnp.exp(lg - jnp.max(lg, axis=1, keepdims=True))
            iota_n = lax.broadcasted_iota(jnp.int32, (G, N), 1)
            col = lax.broadcasted_iota(jnp.int32, (G, K), 1)
            idx_acc = jnp.zeros((G, K), jnp.int32)
            for k in range(K):
                mx = jnp.max(l, axis=1, keepdims=True)
                am = jnp.min(jnp.where(l >= mx, iota_n, N), axis=1,
                             keepdims=True)
                idx_acc = jnp.where(col == k, am, idx_acc)
                l = jnp.where(iota_n == am, -1.0, l)
            idx_ref[d, g * G:(g + 1) * G, :] = idx_acc
            flat_ref[d, g * G:(g + 1) * G, :] = idx_acc + bt * N


def _topk_call(nf, emb):
    two, B, T, N, D = nf.shape
    M = emb.shape[0]
    BT = B * T
    TH = T // 2
    return pl.pallas_call(
        functools.partial(_topk_body, T, N),
        grid=(BT // 2,),
        in_specs=[
            pl.BlockSpec((2, 1, 2, N, D),
                         lambda i: (0, i // TH, i % TH, 0, 0)),
            pl.BlockSpec((M, 2 * D), lambda i: (0, 0)),
        ],
        out_specs=[
            pl.BlockSpec((2, M, K), lambda i: (i, 0, 0)),
            pl.BlockSpec((2, M, K), lambda i: (i, 0, 0)),
        ],
        out_shape=[
            jax.ShapeDtypeStruct((BT, M, K), jnp.int32),
            jax.ShapeDtypeStruct((BT, M, K), jnp.int32),
        ],
    )(nf, emb)


_NW = 32
_CH = 128


def _make_sc_gather(total_rows, D):
    per_w = total_rows // _NW
    nch = per_w // _CH
    mesh = plsc.VectorSubcoreMesh(core_axis_name="c", subcore_axis_name="s")

    @functools.partial(
        pl.kernel,
        out_type=jax.ShapeDtypeStruct((total_rows, D), jnp.float32),
        mesh=mesh,
        scratch_types=[
            pltpu.VMEM((nch, _CH), jnp.int32),
            pltpu.VMEM((_CH, D), jnp.float32),
            pltpu.VMEM((_CH, D), jnp.float32),
            pltpu.SemaphoreType.DMA,
            pltpu.SemaphoreType.DMA,
        ],
    )
    def gather(idx_hbm, table_hbm, out_hbm, idx_v, buf0, buf1, sem0, sem1):
        wid = lax.axis_index("s") * 2 + lax.axis_index("c")
        pltpu.sync_copy(idx_hbm.at[wid], idx_v)
        base = wid * per_w

        pltpu.async_copy(table_hbm.at[idx_v.at[0]], buf0, sem0)

        def step(j, carry):
            c0 = 2 * j
            pltpu.async_copy(table_hbm.at[idx_v.at[c0 + 1]], buf1, sem1)
            pltpu.make_async_copy(table_hbm.at[idx_v.at[c0]], buf0, sem0).wait()
            pltpu.sync_copy(buf0, out_hbm.at[pl.ds(base + c0 * _CH, _CH)])

            @pl.when(j < nch // 2 - 1)
            def _():
                pltpu.async_copy(table_hbm.at[idx_v.at[c0 + 2]], buf0, sem0)

            pltpu.make_async_copy(
                table_hbm.at[idx_v.at[c0 + 1]], buf1, sem1).wait()
            pltpu.sync_copy(buf1,
                            out_hbm.at[pl.ds(base + (c0 + 1) * _CH, _CH)])
            return carry

        lax.fori_loop(0, nch // 2, step, 0)

    return gather


def kernel(node_feature, node_embeddings):
    two, B, T, N, D = node_feature.shape
    M = node_embeddings.shape[0]

    idx, flat1 = _topk_call(node_feature, node_embeddings)
    flat2 = flat1 + B * T * N
    flat = jnp.concatenate([flat1.reshape(-1), flat2.reshape(-1)])
    total_rows = flat.shape[0]

    table = node_feature.reshape(two * B * T * N, D)
    rows = _make_sc_gather(total_rows, D)(
        flat.reshape(_NW, total_rows // (_NW * _CH), _CH), table)
    sel = rows.reshape(2, B, T, M, K, D)

    indices = idx.reshape(B, T, M, K)
    batch_indices = jnp.broadcast_to(
        jnp.arange(B, dtype=indices.dtype).reshape(B, 1, 1, 1), (B, T, M, K))
    time_indices = jnp.broadcast_to(
        jnp.arange(T, dtype=indices.dtype).reshape(1, T, 1, 1), (B, T, M, K))
    return (sel[0], sel[1], batch_indices, time_indices, indices)


# speedup vs baseline: 1.5168x; 1.5124x over previous
import functools

import jax
import jax.numpy as jnp
from jax import lax
from jax.experimental import pallas as pl
from jax.experimental.pallas import tpu as pltpu
from jax.experimental.pallas import tpu_sc as plsc

K = 16


def _topk_body(T, N, nf_ref, emb_ref, idx_ref, flat_ref):
    pid = pl.program_id(0)
    states = [_topk_prepare(N, d, nf_ref, emb_ref) for d in range(2)]
    nrows = [[], []]
    M = 32
    for k in range(K):
        for d in range(2):
            _, Vt, Nt = states[d]
            nrows[d].append(_select_step(M, Vt, Nt))
    for d in range(2):
        _topk_emit(N, d, 2 * pid + d, states[d], nrows[d], idx_ref, flat_ref)


def _topk_prepare(N, d, nf_ref, emb_ref):
    nv1 = nf_ref[0, 0, d]
    nv2 = nf_ref[1, 0, d]
    nv3 = jnp.concatenate([nv1, nv2], axis=-1)
    emb = emb_ref[...]
    logits = lax.dot_general(emb, nv3, (((1,), (1,)), ((), ())))

    M = logits.shape[0]
    C = 128
    R = 4
    NCH = N // C
    BIGN = jnp.int32(1 << 20)

    lane = lax.broadcasted_iota(jnp.int32, (M, C), 1)
    sentv = jnp.full((M, C), -1.0, jnp.float32)
    sentn = jnp.full((M, C), BIGN, jnp.int32)
    mxr = logits[:, 0:C]
    for c in range(1, NCH):
        mxr = jnp.maximum(mxr, logits[:, c * C:(c + 1) * C])
    for s in (64, 32, 16, 8, 4, 2, 1):
        mxr = jnp.maximum(mxr, jnp.roll(mxr, s, axis=1))
    vs = [sentv] * R
    ns = [sentn] * R
    for c in range(NCH):
        nc = lane + c * C
        vc = jnp.exp(logits[:, c * C:(c + 1) * C] - mxr)
        bs = [vc > v for v in vs]
        for r in range(R - 1, 0, -1):
            vs[r] = jnp.where(bs[r - 1], vs[r - 1],
                              jnp.where(bs[r], vc, vs[r]))
            ns[r] = jnp.where(bs[r - 1], ns[r - 1],
                              jnp.where(bs[r], nc, ns[r]))
        vs[0] = jnp.where(bs[0], vc, vs[0])
        ns[0] = jnp.where(bs[0], nc, ns[0])

    V = jnp.concatenate([jnp.transpose(v) for v in vs], axis=1)
    Nn = jnp.concatenate([jnp.transpose(n) for n in ns], axis=1)
    S = 8
    NT = C // S
    Vt = [V[i * S:(i + 1) * S] for i in range(NT)]
    Nt = [Nn[i * S:(i + 1) * S] for i in range(NT)]
    return logits, Vt, Nt


def _select_step(M, Vt, Nt):
    NT = len(Vt)
    mv, mi = Vt[0], Nt[0]
    for i in range(1, NT):
        take = (Vt[i] > mv) | ((Vt[i] == mv) & (Nt[i] < mi))
        mv = jnp.where(take, Vt[i], mv)
        mi = jnp.where(take, Nt[i], mi)
    for ax, s in ((0, 4), (0, 2), (0, 1), (1, M), (1, 2 * M)):
        rv = jnp.roll(mv, s, axis=ax)
        ri = jnp.roll(mi, s, axis=ax)
        take = (rv > mv) | ((rv == mv) & (ri < mi))
        mv = jnp.where(take, rv, mv)
        mi = jnp.where(take, ri, mi)
    nstar = mi
    for i in range(NT):
        Vt[i] = jnp.where(Nt[i] == nstar, -1.0, Vt[i])
    return nstar


def _topk_emit(N, d, bt, state, nrows, idx_ref, flat_ref):
    logits, Vt, Nt = state
    M = logits.shape[0]
    S = 8
    NT = len(Vt)
    R = 4
    BIGN = jnp.int32(1 << 20)
    nmat = jnp.concatenate([nr[0:1] for nr in nrows], axis=0)
    idx_acc = jnp.transpose(nmat[:, :M])
    idx_ref[d] = idx_acc
    flat_ref[d] = idx_acc + bt * N

    exh = jnp.zeros((S, 4 * M), jnp.int32)
    for i in range(NT):
        used = jnp.where(Vt[i] < 0.0, 1, 0)
        u2 = used + jnp.roll(used, M, axis=1)
        u4 = u2 + jnp.roll(u2, 2 * M, axis=1)
        exh = jnp.maximum(exh, u4)
    exhausted = jnp.max(jnp.where(exh >= R, 1, 0))

    G = 8
    @pl.when(exhausted > 0)
    def _slow_path():
        for g in range(M // G):
            lg = logits[g * G:(g + 1) * G, :]
            l = jnp.exp(lg - jnp.max(lg, axis=1, keepdims=True))
            iota_n = lax.broadcasted_iota(jnp.int32, (G, N), 1)
            col = lax.broadcasted_iota(jnp.int32, (G, K), 1)
            idx_acc = jnp.zeros((G, K), jnp.int32)
            for k in range(K):
                mx = jnp.max(l, axis=1, keepdims=True)
                am = jnp.min(jnp.where(l >= mx, iota_n, N), axis=1,
                             keepdims=True)
                idx_acc = jnp.where(col == k, am, idx_acc)
                l = jnp.where(iota_n == am, -1.0, l)
            idx_ref[d, g * G:(g + 1) * G, :] = idx_acc
            flat_ref[d, g * G:(g + 1) * G, :] = idx_acc + bt * N


def _topk_call(nf, emb):
    two, B, T, N, D = nf.shape
    M = emb.shape[0]
    BT = B * T
    TH = T // 2
    return pl.pallas_call(
        functools.partial(_topk_body, T, N),
        grid=(BT // 2,),
        in_specs=[
            pl.BlockSpec((2, 1, 2, N, D),
                         lambda i: (0, i // TH, i % TH, 0, 0)),
            pl.BlockSpec((M, 2 * D), lambda i: (0, 0)),
        ],
        out_specs=[
            pl.BlockSpec((2, M, K), lambda i: (i, 0, 0)),
            pl.BlockSpec((2, M, K), lambda i: (i, 0, 0)),
        ],
        out_shape=[
            jax.ShapeDtypeStruct((BT, M, K), jnp.int32),
            jax.ShapeDtypeStruct((BT, M, K), jnp.int32),
        ],
    )(nf, emb)


_NW = 32
_CH = 128


def _make_sc_gather(total_rows, D):
    per_w = total_rows // _NW
    nch = per_w // _CH
    mesh = plsc.VectorSubcoreMesh(core_axis_name="c", subcore_axis_name="s")

    @functools.partial(
        pl.kernel,
        out_type=jax.ShapeDtypeStruct((total_rows, D), jnp.float32),
        mesh=mesh,
        scratch_types=[
            pltpu.VMEM((nch, _CH), jnp.int32),
            pltpu.VMEM((_CH, D), jnp.float32),
            pltpu.VMEM((_CH, D), jnp.float32),
            pltpu.SemaphoreType.DMA,
            pltpu.SemaphoreType.DMA,
        ],
    )
    def gather(idx_hbm, table_hbm, out_hbm, idx_v, buf0, buf1, sem0, sem1):
        wid = lax.axis_index("s") * 2 + lax.axis_index("c")
        pltpu.sync_copy(idx_hbm.at[wid], idx_v)
        base = wid * per_w

        pltpu.async_copy(table_hbm.at[idx_v.at[0]], buf0, sem0)

        def step(j, carry):
            c0 = 2 * j
            pltpu.async_copy(table_hbm.at[idx_v.at[c0 + 1]], buf1, sem1)
            pltpu.make_async_copy(table_hbm.at[idx_v.at[c0]], buf0, sem0).wait()
            pltpu.sync_copy(buf0, out_hbm.at[pl.ds(base + c0 * _CH, _CH)])

            @pl.when(j < nch // 2 - 1)
            def _():
                pltpu.async_copy(table_hbm.at[idx_v.at[c0 + 2]], buf0, sem0)

            pltpu.make_async_copy(
                table_hbm.at[idx_v.at[c0 + 1]], buf1, sem1).wait()
            pltpu.sync_copy(buf1,
                            out_hbm.at[pl.ds(base + (c0 + 1) * _CH, _CH)])
            return carry

        lax.fori_loop(0, nch // 2, step, 0)

    return gather


def kernel(node_feature, node_embeddings):
    two, B, T, N, D = node_feature.shape
    M = node_embeddings.shape[0]

    idx, flat1 = _topk_call(node_feature, node_embeddings)
    flat2 = flat1 + B * T * N
    flat = jnp.concatenate([flat1.reshape(-1), flat2.reshape(-1)])
    total_rows = flat.shape[0]

    table = node_feature.reshape(two * B * T * N, D)
    rows = _make_sc_gather(total_rows, D)(
        flat.reshape(_NW, total_rows // (_NW * _CH), _CH), table)
    sel = rows.reshape(2, B, T, M, K, D)

    indices = idx.reshape(B, T, M, K)
    batch_indices = jnp.broadcast_to(
        jnp.arange(B, dtype=indices.dtype).reshape(B, 1, 1, 1), (B, T, M, K))
    time_indices = jnp.broadcast_to(
        jnp.arange(T, dtype=indices.dtype).reshape(1, T, 1, 1), (B, T, M, K))
    return (sel[0], sel[1], batch_indices, time_indices, indices)


# speedup vs baseline: 1.9410x; 1.2797x over previous
import functools

import jax
import jax.numpy as jnp
from jax import lax
from jax.experimental import pallas as pl
from jax.experimental.pallas import tpu as pltpu
from jax.experimental.pallas import tpu_sc as plsc

K = 16


def _topk_body(T, N, nf_ref, emb_ref, idx_ref, flat_ref):
    pid = pl.program_id(0)
    states = [_topk_prepare(N, d, nf_ref, emb_ref) for d in range(4)]
    nrows = [[], [], [], []]
    M = 32
    for k in range(K):
        for d in range(4):
            _, Vt, Nt = states[d]
            nrows[d].append(_select_step(M, Vt, Nt))
    for d in range(4):
        _topk_emit(N, d, 4 * pid + d, states[d], nrows[d], idx_ref, flat_ref)


def _topk_prepare(N, d, nf_ref, emb_ref):
    nv1 = nf_ref[0, 0, d]
    nv2 = nf_ref[1, 0, d]
    nv3 = jnp.concatenate([nv1, nv2], axis=-1)
    emb = emb_ref[...]
    logits = lax.dot_general(emb, nv3, (((1,), (1,)), ((), ())))

    M = logits.shape[0]
    C = 128
    R = 4
    NCH = N // C
    BIGN = jnp.int32(1 << 20)

    lane = lax.broadcasted_iota(jnp.int32, (M, C), 1)
    sentv = jnp.full((M, C), -1.0, jnp.float32)
    sentn = jnp.full((M, C), BIGN, jnp.int32)
    mxr = logits[:, 0:C]
    for c in range(1, NCH):
        mxr = jnp.maximum(mxr, logits[:, c * C:(c + 1) * C])
    for s in (64, 32, 16, 8, 4, 2, 1):
        mxr = jnp.maximum(mxr, jnp.roll(mxr, s, axis=1))
    vs = [sentv] * R
    ns = [sentn] * R
    for c in range(NCH):
        nc = lane + c * C
        vc = jnp.exp(logits[:, c * C:(c + 1) * C] - mxr)
        bs = [vc > v for v in vs]
        for r in range(R - 1, 0, -1):
            vs[r] = jnp.where(bs[r - 1], vs[r - 1],
                              jnp.where(bs[r], vc, vs[r]))
            ns[r] = jnp.where(bs[r - 1], ns[r - 1],
                              jnp.where(bs[r], nc, ns[r]))
        vs[0] = jnp.where(bs[0], vc, vs[0])
        ns[0] = jnp.where(bs[0], nc, ns[0])

    V = jnp.concatenate([jnp.transpose(v) for v in vs], axis=1)
    Nn = jnp.concatenate([jnp.transpose(n) for n in ns], axis=1)
    S = 8
    NT = C // S
    Vt = [V[i * S:(i + 1) * S] for i in range(NT)]
    Nt = [Nn[i * S:(i + 1) * S] for i in range(NT)]
    return logits, Vt, Nt


def _select_step(M, Vt, Nt):
    NT = len(Vt)
    mv, mi = Vt[0], Nt[0]
    for i in range(1, NT):
        take = (Vt[i] > mv) | ((Vt[i] == mv) & (Nt[i] < mi))
        mv = jnp.where(take, Vt[i], mv)
        mi = jnp.where(take, Nt[i], mi)
    for ax, s in ((0, 4), (0, 2), (0, 1), (1, M), (1, 2 * M)):
        rv = jnp.roll(mv, s, axis=ax)
        ri = jnp.roll(mi, s, axis=ax)
        take = (rv > mv) | ((rv == mv) & (ri < mi))
        mv = jnp.where(take, rv, mv)
        mi = jnp.where(take, ri, mi)
    nstar = mi
    for i in range(NT):
        Vt[i] = jnp.where(Nt[i] == nstar, -1.0, Vt[i])
    return nstar


def _topk_emit(N, d, bt, state, nrows, idx_ref, flat_ref):
    logits, Vt, Nt = state
    M = logits.shape[0]
    S = 8
    NT = len(Vt)
    R = 4
    BIGN = jnp.int32(1 << 20)
    nmat = jnp.concatenate([nr[0:1] for nr in nrows], axis=0)
    idx_acc = jnp.transpose(nmat[:, :M])
    idx_ref[d] = idx_acc
    flat_ref[d] = idx_acc + bt * N

    exh = jnp.zeros((S, 4 * M), jnp.int32)
    for i in range(NT):
        used = jnp.where(Vt[i] < 0.0, 1, 0)
        u2 = used + jnp.roll(used, M, axis=1)
        u4 = u2 + jnp.roll(u2, 2 * M, axis=1)
        exh = jnp.maximum(exh, u4)
    exhausted = jnp.max(jnp.where(exh >= R, 1, 0))

    G = 8
    @pl.when(exhausted > 0)
    def _slow_path():
        for g in range(M // G):
            lg = logits[g * G:(g + 1) * G, :]
            l = jnp.exp(lg - jnp.max(lg, axis=1, keepdims=True))
            iota_n = lax.broadcasted_iota(jnp.int32, (G, N), 1)
            col = lax.broadcasted_iota(jnp.int32, (G, K), 1)
            idx_acc = jnp.zeros((G, K), jnp.int32)
            for k in range(K):
                mx = jnp.max(l, axis=1, keepdims=True)
                am = jnp.min(jnp.where(l >= mx, iota_n, N), axis=1,
                             keepdims=True)
                idx_acc = jnp.where(col == k, am, idx_acc)
                l = jnp.where(iota_n == am, -1.0, l)
            idx_ref[d, g * G:(g + 1) * G, :] = idx_acc
            flat_ref[d, g * G:(g + 1) * G, :] = idx_acc + bt * N


def _topk_call(nf, emb):
    two, B, T, N, D = nf.shape
    M = emb.shape[0]
    BT = B * T
    TH = T // 4
    return pl.pallas_call(
        functools.partial(_topk_body, T, N),
        grid=(BT // 4,),
        in_specs=[
            pl.BlockSpec((2, 1, 4, N, D),
                         lambda i: (0, i // TH, i % TH, 0, 0)),
            pl.BlockSpec((M, 2 * D), lambda i: (0, 0)),
        ],
        out_specs=[
            pl.BlockSpec((4, M, K), lambda i: (i, 0, 0)),
            pl.BlockSpec((4, M, K), lambda i: (i, 0, 0)),
        ],
        out_shape=[
            jax.ShapeDtypeStruct((BT, M, K), jnp.int32),
            jax.ShapeDtypeStruct((BT, M, K), jnp.int32),
        ],
    )(nf, emb)


_NW = 32
_CH = 128


def _make_sc_gather(total_rows, D):
    per_w = total_rows // _NW
    nch = per_w // _CH
    mesh = plsc.VectorSubcoreMesh(core_axis_name="c", subcore_axis_name="s")

    @functools.partial(
        pl.kernel,
        out_type=jax.ShapeDtypeStruct((total_rows, D), jnp.float32),
        mesh=mesh,
        scratch_types=[
            pltpu.VMEM((nch, _CH), jnp.int32),
            pltpu.VMEM((_CH, D), jnp.float32),
            pltpu.VMEM((_CH, D), jnp.float32),
            pltpu.SemaphoreType.DMA,
            pltpu.SemaphoreType.DMA,
        ],
    )
    def gather(idx_hbm, table_hbm, out_hbm, idx_v, buf0, buf1, sem0, sem1):
        wid = lax.axis_index("s") * 2 + lax.axis_index("c")
        pltpu.sync_copy(idx_hbm.at[wid], idx_v)
        base = wid * per_w

        pltpu.async_copy(table_hbm.at[idx_v.at[0]], buf0, sem0)

        def step(j, carry):
            c0 = 2 * j
            pltpu.async_copy(table_hbm.at[idx_v.at[c0 + 1]], buf1, sem1)
            pltpu.make_async_copy(table_hbm.at[idx_v.at[c0]], buf0, sem0).wait()
            pltpu.sync_copy(buf0, out_hbm.at[pl.ds(base + c0 * _CH, _CH)])

            @pl.when(j < nch // 2 - 1)
            def _():
                pltpu.async_copy(table_hbm.at[idx_v.at[c0 + 2]], buf0, sem0)

            pltpu.make_async_copy(
                table_hbm.at[idx_v.at[c0 + 1]], buf1, sem1).wait()
            pltpu.sync_copy(buf1,
                            out_hbm.at[pl.ds(base + (c0 + 1) * _CH, _CH)])
            return carry

        lax.fori_loop(0, nch // 2, step, 0)

    return gather


def kernel(node_feature, node_embeddings):
    two, B, T, N, D = node_feature.shape
    M = node_embeddings.shape[0]

    idx, flat1 = _topk_call(node_feature, node_embeddings)
    flat2 = flat1 + B * T * N
    flat = jnp.concatenate([flat1.reshape(-1), flat2.reshape(-1)])
    total_rows = flat.shape[0]

    table = node_feature.reshape(two * B * T * N, D)
    rows = _make_sc_gather(total_rows, D)(
        flat.reshape(_NW, total_rows // (_NW * _CH), _CH), table)
    sel = rows.reshape(2, B, T, M, K, D)

    indices = idx.reshape(B, T, M, K)
    batch_indices = jnp.broadcast_to(
        jnp.arange(B, dtype=indices.dtype).reshape(B, 1, 1, 1), (B, T, M, K))
    time_indices = jnp.broadcast_to(
        jnp.arange(T, dtype=indices.dtype).reshape(1, T, 1, 1), (B, T, M, K))
    return (sel[0], sel[1], batch_indices, time_indices, indices)
